# Initial kernel scaffold; baseline (speedup 1.0000x reference)
import jax
import jax.numpy as jnp
from jax.experimental import pallas as pl


def kernel(res_feat, meta_feat, sec_ids, batch_pe_vector, batch_meta_2_node_edge, batch_meta_2_node_vector, g1_w1, g1_b1, g1_w2, g1_b2, g2_w1, g2_b1, g2_w2, g2_b2, f_w1, f_b1, f_w2, f_b2):
    raise NotImplementedError("write your pallas kernel here")


# speedup vs baseline: 2.1539x; 2.1539x over previous
import functools

import jax
import jax.numpy as jnp
from jax import lax
from jax.experimental import pallas as pl
from jax.experimental.pallas import tpu as pltpu
from jax.experimental.pallas import tpu_sc as plsc

F32 = jnp.float32
I32 = jnp.int32

NC = 2
NS = 16
NW = NC * NS
CH = 128
SCH = 64


def _ceil_to(x, m):
    return (x + m - 1) // m * m


def _tc1_body(res_ref, meta_ref, w2a_ref, w1a_ref, w1b_ref,
              pres_ref, qres_ref, tsec_ref):
    D = res_ref.shape[1]
    res = res_ref[...]
    meta = meta_ref[...]
    pres_ref[...] = jnp.dot(res, w2a_ref[...], preferred_element_type=F32)
    qres_ref[...] = jnp.dot(res, w1a_ref[...], preferred_element_type=F32)
    tsec_ref[:, :D] = jnp.dot(meta, w1b_ref[...], preferred_element_type=F32)
    tsec_ref[:, D:] = meta


def _tc2_body(g1_ref, mg_ref, vec_ref, w2b_ref, wv_ref, b1_ref, w2_ref, b2_ref,
              out_ref):
    mg = mg_ref[...]
    h = (g1_ref[...] + jnp.dot(mg, w2b_ref[...], preferred_element_type=F32)
         + jnp.dot(vec_ref[...], wv_ref[...], preferred_element_type=F32)
         + b1_ref[...])
    h = jnp.maximum(h, 0.0)
    g = jnp.dot(h, w2_ref[...], preferred_element_type=F32) + b2_ref[...]
    out_ref[...] = g * mg


def _tc3_body(res_ref, gsec_ref, qres_ref, pe_ref, ps_ref, pc_ref,
              w1v_ref, b1g_ref, w1w2_ref, b2g_ref,
              fw1_ref, fb1_ref, fw2_ref, fb2_ref, out_ref):
    D = res_ref.shape[1]
    gsec = gsec_ref[...]
    u = (qres_ref[...] + gsec[:, :D]
         + jnp.dot(pe_ref[...], w1v_ref[...], preferred_element_type=F32)
         + b1g_ref[...])
    u = jnp.maximum(u, 0.0)
    gate1 = jnp.dot(u, w1w2_ref[...], preferred_element_type=F32) + b2g_ref[...]
    sums = ps_ref[0] + ps_ref[1]
    cnt = pc_ref[0, :, 0:1] + pc_ref[1, :, 0:1]
    mean = sums / jnp.maximum(cnt, 1.0)
    fused = res_ref[...] + gate1 * gsec[:, D:] + mean
    hh = jnp.maximum(
        jnp.dot(fused, fw1_ref[...], preferred_element_type=F32) + fb1_ref[...], 0.0)
    out_ref[...] = jnp.dot(hh, fw2_ref[...], preferred_element_type=F32) + fb2_ref[...]


def _make_sc_gather(Epad, EPW, Np, NPW, D):
    mesh = plsc.VectorSubcoreMesh(core_axis_name="c", subcore_axis_name="s")

    @functools.partial(
        pl.kernel, mesh=mesh,
        out_type=(
            jax.ShapeDtypeStruct((Epad, D), F32),
            jax.ShapeDtypeStruct((Epad, D), F32),
            jax.ShapeDtypeStruct((Np, 2 * D), F32),
        ),
        scratch_types=[
            pltpu.VMEM((CH,), I32),
            pltpu.VMEM((CH,), I32),
            pltpu.VMEM((SCH,), I32),
            pltpu.VMEM((CH, D), F32),
            pltpu.VMEM((CH, D), F32),
            pltpu.VMEM((SCH, 2 * D), F32),
            pltpu.SemaphoreType.DMA,
            pltpu.SemaphoreType.DMA,
        ],
    )
    def sc_gather(pres_hbm, tsrc_hbm, tsec_hbm, src_hbm, dst_hbm, sec_hbm,
                  g1_hbm, g2_hbm, gsec_hbm,
                  idx_d, idx_s, idx_n, bufa, bufb, bufn, sem1, sem2):
        wid = lax.axis_index("s") * NC + lax.axis_index("c")
        ebase = wid * EPW

        def edge_chunk(k, carry):
            off = pl.multiple_of(ebase + k * CH, CH)
            pltpu.sync_copy(dst_hbm.at[pl.ds(off, CH)], idx_d)
            pltpu.sync_copy(src_hbm.at[pl.ds(off, CH)], idx_s)
            cp1 = pltpu.async_copy(pres_hbm.at[idx_d], bufa, sem1)
            cp2 = pltpu.async_copy(tsrc_hbm.at[idx_s], bufb, sem2)
            cp1.wait()
            cp2.wait()
            pltpu.sync_copy(bufa, g1_hbm.at[pl.ds(off, CH)])
            pltpu.sync_copy(bufb, g2_hbm.at[pl.ds(off, CH)])
            return carry

        lax.fori_loop(0, EPW // CH, edge_chunk, 0)

        nbase = wid * NPW

        def sec_chunk(j, carry):
            off = pl.multiple_of(nbase + j * SCH, SCH)
            pltpu.sync_copy(sec_hbm.at[pl.ds(off, SCH)], idx_n)
            pltpu.async_copy(tsec_hbm.at[idx_n], bufn, sem1).wait()
            pltpu.sync_copy(bufn, gsec_hbm.at[pl.ds(off, SCH)])
            return carry

        lax.fori_loop(0, NPW // SCH, sec_chunk, 0)

    return sc_gather


def _make_sc_scatter(Epad, EPW, NA, D):
    mesh = plsc.VectorSubcoreMesh(core_axis_name="c", subcore_axis_name="s")
    RPS = NA // NS
    CW = 16

    @functools.partial(
        pl.kernel, mesh=mesh,
        compiler_params=pltpu.CompilerParams(use_tc_tiling_on_sc=False),
        out_type=(
            jax.ShapeDtypeStruct((NC, NA, D), F32),
            jax.ShapeDtypeStruct((NC, NA, CW), F32),
        ),
        scratch_types=[
            pltpu.VMEM_SHARED((NA, D), F32),
            pltpu.VMEM_SHARED((NA, CW), F32),
            pltpu.VMEM((CH,), I32),
            pltpu.VMEM((CH, D), F32),
            pltpu.VMEM((CH, CW), F32),
        ],
    )
    def sc_scatter(contrib_hbm, dst_hbm, zsum_hbm, zcnt_hbm, ones_hbm,
                   sums_hbm, cnts_hbm, acc, cacc, idx, buf, ones):
        c = lax.axis_index("c")
        s = lax.axis_index("s")
        wid = s * NC + c

        @pl.when(s == 0)
        def _():
            pltpu.sync_copy(zsum_hbm, acc)
            pltpu.sync_copy(zcnt_hbm, cacc)

        pltpu.sync_copy(ones_hbm, ones)
        plsc.subcore_barrier()

        ebase = wid * EPW

        def edge_chunk(k, carry):
            off = pl.multiple_of(ebase + k * CH, CH)
            pltpu.sync_copy(dst_hbm.at[pl.ds(off, CH)], idx)
            pltpu.sync_copy(contrib_hbm.at[pl.ds(off, CH)], buf)
            pltpu.sync_copy(buf, acc.at[idx], add=True)
            pltpu.sync_copy(ones, cacc.at[idx], add=True)
            return carry

        lax.fori_loop(0, EPW // CH, edge_chunk, 0)
        plsc.subcore_barrier()

        r0 = s * RPS
        for j in range(RPS // CH):
            row = r0 + j * CH
            pltpu.sync_copy(acc.at[pl.ds(row, CH)], sums_hbm.at[c, pl.ds(row, CH)])
            pltpu.sync_copy(cacc.at[pl.ds(row, CH)], cnts_hbm.at[c, pl.ds(row, CH)])

    return sc_scatter


def kernel(res_feat, meta_feat, sec_ids, batch_pe_vector, batch_meta_2_node_edge,
           batch_meta_2_node_vector, g1_w1, g1_b1, g1_w2, g1_b2,
           g2_w1, g2_b1, g2_w2, g2_b2, f_w1, f_b1, f_w2, f_b2):
    N, D = res_feat.shape
    M = meta_feat.shape[0]
    E = batch_meta_2_node_edge.shape[1]
    assert D % 16 == 0

    EPW = _ceil_to(-(-E // NW), CH)
    Epad = EPW * NW
    NPW = _ceil_to(-(-N // NW), SCH)
    Np = NPW * NW
    NA = Np
    assert NA % (NS * CH) == 0 and NA >= N + 1 and NA >= M
    BN = 512
    assert Np % BN == 0 and Epad % BN == 0

    edges = batch_meta_2_node_edge.astype(I32)
    src_p = jnp.pad(edges[0], (0, Epad - E))
    dst_p = jnp.pad(edges[1], (0, Epad - E), constant_values=N)
    vec_p = jnp.zeros((Epad, 8), F32).at[:E, :3].set(batch_meta_2_node_vector)
    sec_p = jnp.pad(sec_ids.astype(I32), (0, Np - N))
    res_p = jnp.pad(res_feat, ((0, Np - N), (0, 0)))
    meta_p = jnp.pad(meta_feat, ((0, Np - M), (0, 0)))
    pe_p = jnp.zeros((Np, 8), F32).at[:N, :3].set(batch_pe_vector)
    w2v_p = jnp.zeros((8, D), F32).at[:3].set(g2_w1[2 * D:])
    w1v_n = jnp.zeros((8, D), F32).at[:3].set(-g1_w1[2 * D:])
    b1g = g1_b1.reshape(1, D)
    b2g = g1_b2.reshape(1, D)
    b1e = g2_b1.reshape(1, D)
    b2e = g2_b2.reshape(1, D)
    fb1 = f_b1.reshape(1, D)
    fb2 = f_b2.reshape(1, D)

    cparams = pltpu.CompilerParams(dimension_semantics=("arbitrary",))

    full = lambda i: (0, 0)
    row_blk = pl.BlockSpec((BN, D), lambda i: (i, 0))
    row_blk2 = pl.BlockSpec((BN, 2 * D), lambda i: (i, 0))
    wspec = pl.BlockSpec((D, D), full)
    p_res, q_res, t_sec = pl.pallas_call(
        _tc1_body,
        grid=(Np // BN,),
        in_specs=[row_blk, row_blk, wspec, wspec, wspec],
        out_specs=[row_blk, row_blk, row_blk2],
        out_shape=[
            jax.ShapeDtypeStruct((Np, D), F32),
            jax.ShapeDtypeStruct((Np, D), F32),
            jax.ShapeDtypeStruct((Np, 2 * D), F32),
        ],
        compiler_params=cparams,
    )(res_p, meta_p, g2_w1[:D], g1_w1[:D], g1_w1[D:2 * D])

    sc_gather = _make_sc_gather(Epad, EPW, Np, NPW, D)
    g1e, mge, gsec = sc_gather(p_res, meta_p, t_sec, src_p, dst_p, sec_p)

    contrib = pl.pallas_call(
        _tc2_body,
        grid=(Epad // BN,),
        in_specs=[
            row_blk, row_blk,
            pl.BlockSpec((BN, 8), lambda i: (i, 0)),
            wspec,
            pl.BlockSpec((8, D), full),
            pl.BlockSpec((1, D), full),
            wspec,
            pl.BlockSpec((1, D), full),
        ],
        out_specs=row_blk,
        out_shape=jax.ShapeDtypeStruct((Epad, D), F32),
        compiler_params=cparams,
    )(g1e, mge, vec_p, g2_w1[D:2 * D], w2v_p, b1e, g2_w2, b2e)

    sc_scatter = _make_sc_scatter(Epad, EPW, NA, D)
    zsum = jnp.zeros((NA, D), F32)
    zcnt = jnp.zeros((NA, 16), F32)
    ones_rows = jnp.ones((CH, 16), F32)
    psums, pcnts = sc_scatter(contrib, dst_p, zsum, zcnt, ones_rows)

    out_p = pl.pallas_call(
        _tc3_body,
        grid=(Np // BN,),
        in_specs=[
            row_blk, row_blk2, row_blk,
            pl.BlockSpec((BN, 8), lambda i: (i, 0)),
            pl.BlockSpec((NC, BN, D), lambda i: (0, i, 0)),
            pl.BlockSpec((NC, BN, 16), lambda i: (0, i, 0)),
            pl.BlockSpec((8, D), full),
            pl.BlockSpec((1, D), full),
            wspec,
            pl.BlockSpec((1, D), full),
            wspec,
            pl.BlockSpec((1, D), full),
            wspec,
            pl.BlockSpec((1, D), full),
        ],
        out_specs=row_blk,
        out_shape=jax.ShapeDtypeStruct((Np, D), F32),
        compiler_params=cparams,
    )(res_p, gsec, q_res, pe_p, psums, pcnts,
      w1v_n, b1g, g1_w2, b2g, f_w1, fb1, f_w2, fb2)

    return out_p[:N]


# speedup vs baseline: 2.3918x; 1.1104x over previous
import functools

import jax
import jax.numpy as jnp
from jax import lax
from jax.experimental import pallas as pl
from jax.experimental.pallas import tpu as pltpu
from jax.experimental.pallas import tpu_sc as plsc

F32 = jnp.float32
I32 = jnp.int32

NC = 2
NS = 16
NW = NC * NS
CH = 128
SCH = 64


def _ceil_to(x, m):
    return (x + m - 1) // m * m


def _tc1_body(res_ref, meta_ref, w2a_ref, w1a_ref, w1b_ref,
              pres_ref, qres_ref, tsec_ref):
    D = res_ref.shape[1]
    res = res_ref[...]
    meta = meta_ref[...]
    pres_ref[...] = jnp.dot(res, w2a_ref[...], preferred_element_type=F32)
    qres_ref[...] = jnp.dot(res, w1a_ref[...], preferred_element_type=F32)
    tsec_ref[:, :D] = jnp.dot(meta, w1b_ref[...], preferred_element_type=F32)
    tsec_ref[:, D:] = meta


def _tc2_body(g1_ref, mg_ref, vec_ref, w2b_ref, wv_ref, b1_ref, w2_ref, b2_ref,
              out_ref):
    mg = mg_ref[...]
    h = (g1_ref[...] + jnp.dot(mg, w2b_ref[...], preferred_element_type=F32)
         + jnp.dot(vec_ref[...], wv_ref[...], preferred_element_type=F32)
         + b1_ref[...])
    h = jnp.maximum(h, 0.0)
    g = jnp.dot(h, w2_ref[...], preferred_element_type=F32) + b2_ref[...]
    out_ref[...] = g * mg


def _tc3_body(res_ref, gsec_ref, qres_ref, pe_ref, ps_ref, pc_ref,
              w1v_ref, b1g_ref, w1w2_ref, b2g_ref,
              fw1_ref, fb1_ref, fw2_ref, fb2_ref, out_ref):
    D = res_ref.shape[1]
    gsec = gsec_ref[...]
    u = (qres_ref[...] + gsec[:, :D]
         + jnp.dot(pe_ref[...], w1v_ref[...], preferred_element_type=F32)
         + b1g_ref[...])
    u = jnp.maximum(u, 0.0)
    gate1 = jnp.dot(u, w1w2_ref[...], preferred_element_type=F32) + b2g_ref[...]
    sums = ps_ref[0] + ps_ref[1]
    cnt = pc_ref[0, :, 0:1] + pc_ref[1, :, 0:1]
    mean = sums / jnp.maximum(cnt, 1.0)
    fused = res_ref[...] + gate1 * gsec[:, D:] + mean
    hh = jnp.maximum(
        jnp.dot(fused, fw1_ref[...], preferred_element_type=F32) + fb1_ref[...], 0.0)
    out_ref[...] = jnp.dot(hh, fw2_ref[...], preferred_element_type=F32) + fb2_ref[...]


def _make_sc_gather(Epad, EPW, Np, NPW, D):
    mesh = plsc.VectorSubcoreMesh(core_axis_name="c", subcore_axis_name="s")
    TOT = Epad // CH
    PERC = TOT // NS
    N0 = max(1, round(PERC * 0.419))
    N1 = PERC - N0

    @functools.partial(
        pl.kernel, mesh=mesh,
        out_type=(
            jax.ShapeDtypeStruct((Epad, D), F32),
            jax.ShapeDtypeStruct((Epad, D), F32),
            jax.ShapeDtypeStruct((Np, 2 * D), F32),
        ),
        scratch_types=[
            pltpu.VMEM((CH,), I32),
            pltpu.VMEM((CH,), I32),
            pltpu.VMEM((SCH,), I32),
            pltpu.VMEM((CH, D), F32),
            pltpu.VMEM((CH, D), F32),
            pltpu.VMEM((SCH, 2 * D), F32),
            pltpu.SemaphoreType.DMA,
            pltpu.SemaphoreType.DMA,
        ],
    )
    def sc_gather(pres_hbm, tsrc_hbm, tsec_hbm, src_hbm, dst_hbm, sec_hbm,
                  g1_hbm, g2_hbm, gsec_hbm,
                  idx_d, idx_s, idx_n, bufa, bufb, bufn, sem1, sem2):
        c = lax.axis_index("c")
        s = lax.axis_index("s")
        wid = s * NC + c
        is0 = c == 0
        nchunk = jnp.where(is0, N0, N1)
        cstart = jnp.where(is0, s * N0, NS * N0 + s * N1)

        def edge_chunk(k, carry):
            off = pl.multiple_of((cstart + k) * CH, CH)
            pltpu.sync_copy(dst_hbm.at[pl.ds(off, CH)], idx_d)
            pltpu.sync_copy(src_hbm.at[pl.ds(off, CH)], idx_s)
            cp1 = pltpu.async_copy(pres_hbm.at[idx_d], bufa, sem1)
            cp2 = pltpu.async_copy(tsrc_hbm.at[idx_s], bufb, sem2)
            cp1.wait()
            cp2.wait()
            pltpu.sync_copy(bufa, g1_hbm.at[pl.ds(off, CH)])
            pltpu.sync_copy(bufb, g2_hbm.at[pl.ds(off, CH)])
            return carry

        lax.fori_loop(0, nchunk, edge_chunk, 0)

        nbase = wid * NPW

        def sec_chunk(j, carry):
            off = pl.multiple_of(nbase + j * SCH, SCH)
            pltpu.sync_copy(sec_hbm.at[pl.ds(off, SCH)], idx_n)
            pltpu.async_copy(tsec_hbm.at[idx_n], bufn, sem1).wait()
            pltpu.sync_copy(bufn, gsec_hbm.at[pl.ds(off, SCH)])
            return carry

        lax.fori_loop(0, NPW // SCH, sec_chunk, 0)

    return sc_gather


def _make_sc_scatter(Epad, EPW, NA, D):
    mesh = plsc.VectorSubcoreMesh(core_axis_name="c", subcore_axis_name="s")
    RPS = NA // NS
    CW = 16

    @functools.partial(
        pl.kernel, mesh=mesh,
        compiler_params=pltpu.CompilerParams(use_tc_tiling_on_sc=False),
        out_type=(
            jax.ShapeDtypeStruct((NC, NA, D), F32),
            jax.ShapeDtypeStruct((NC, NA, CW), F32),
        ),
        scratch_types=[
            pltpu.VMEM_SHARED((NA, D), F32),
            pltpu.VMEM_SHARED((NA, CW), F32),
            pltpu.VMEM((CH,), I32),
            pltpu.VMEM((CH, D), F32),
            pltpu.VMEM((CH, CW), F32),
        ],
    )
    def sc_scatter(contrib_hbm, dst_hbm, zsum_hbm, zcnt_hbm, ones_hbm,
                   sums_hbm, cnts_hbm, acc, cacc, idx, buf, ones):
        c = lax.axis_index("c")
        s = lax.axis_index("s")
        wid = s * NC + c

        @pl.when(s == 0)
        def _():
            pltpu.sync_copy(zsum_hbm, acc)
            pltpu.sync_copy(zcnt_hbm, cacc)

        pltpu.sync_copy(ones_hbm, ones)
        plsc.subcore_barrier()

        ebase = wid * EPW

        def edge_chunk(k, carry):
            off = pl.multiple_of(ebase + k * CH, CH)
            pltpu.sync_copy(dst_hbm.at[pl.ds(off, CH)], idx)
            pltpu.sync_copy(contrib_hbm.at[pl.ds(off, CH)], buf)
            pltpu.sync_copy(buf, acc.at[idx], add=True)
            pltpu.sync_copy(ones, cacc.at[idx], add=True)
            return carry

        lax.fori_loop(0, EPW // CH, edge_chunk, 0)
        plsc.subcore_barrier()

        r0 = s * RPS
        for j in range(RPS // CH):
            row = r0 + j * CH
            pltpu.sync_copy(acc.at[pl.ds(row, CH)], sums_hbm.at[c, pl.ds(row, CH)])
            pltpu.sync_copy(cacc.at[pl.ds(row, CH)], cnts_hbm.at[c, pl.ds(row, CH)])

    return sc_scatter


def kernel(res_feat, meta_feat, sec_ids, batch_pe_vector, batch_meta_2_node_edge,
           batch_meta_2_node_vector, g1_w1, g1_b1, g1_w2, g1_b2,
           g2_w1, g2_b1, g2_w2, g2_b2, f_w1, f_b1, f_w2, f_b2):
    N, D = res_feat.shape
    M = meta_feat.shape[0]
    E = batch_meta_2_node_edge.shape[1]
    assert D % 16 == 0

    EPW = _ceil_to(-(-E // NW), CH)
    Epad = EPW * NW
    NPW = _ceil_to(-(-N // NW), SCH)
    Np = NPW * NW
    NA = Np
    assert NA % (NS * CH) == 0 and NA >= N + 1 and NA >= M
    BN = 512
    assert Np % BN == 0 and Epad % BN == 0

    edges = batch_meta_2_node_edge.astype(I32)
    src_p = jnp.pad(edges[0], (0, Epad - E))
    dst_p = jnp.pad(edges[1], (0, Epad - E), constant_values=N)
    vec_p = jnp.zeros((Epad, 8), F32).at[:E, :3].set(batch_meta_2_node_vector)
    sec_p = jnp.pad(sec_ids.astype(I32), (0, Np - N))
    res_p = jnp.pad(res_feat, ((0, Np - N), (0, 0)))
    meta_p = jnp.pad(meta_feat, ((0, Np - M), (0, 0)))
    pe_p = jnp.zeros((Np, 8), F32).at[:N, :3].set(batch_pe_vector)
    w2v_p = jnp.zeros((8, D), F32).at[:3].set(g2_w1[2 * D:])
    w1v_n = jnp.zeros((8, D), F32).at[:3].set(-g1_w1[2 * D:])
    b1g = g1_b1.reshape(1, D)
    b2g = g1_b2.reshape(1, D)
    b1e = g2_b1.reshape(1, D)
    b2e = g2_b2.reshape(1, D)
    fb1 = f_b1.reshape(1, D)
    fb2 = f_b2.reshape(1, D)

    cparams = pltpu.CompilerParams(dimension_semantics=("arbitrary",))

    full = lambda i: (0, 0)
    row_blk = pl.BlockSpec((BN, D), lambda i: (i, 0))
    row_blk2 = pl.BlockSpec((BN, 2 * D), lambda i: (i, 0))
    wspec = pl.BlockSpec((D, D), full)
    p_res, q_res, t_sec = pl.pallas_call(
        _tc1_body,
        grid=(Np // BN,),
        in_specs=[row_blk, row_blk, wspec, wspec, wspec],
        out_specs=[row_blk, row_blk, row_blk2],
        out_shape=[
            jax.ShapeDtypeStruct((Np, D), F32),
            jax.ShapeDtypeStruct((Np, D), F32),
            jax.ShapeDtypeStruct((Np, 2 * D), F32),
        ],
        compiler_params=cparams,
    )(res_p, meta_p, g2_w1[:D], g1_w1[:D], g1_w1[D:2 * D])

    sc_gather = _make_sc_gather(Epad, EPW, Np, NPW, D)
    g1e, mge, gsec = sc_gather(p_res, meta_p, t_sec, src_p, dst_p, sec_p)

    BE = 1024
    assert Epad % BE == 0
    erow_blk = pl.BlockSpec((BE, D), lambda i: (i, 0))
    contrib = pl.pallas_call(
        _tc2_body,
        grid=(Epad // BE,),
        in_specs=[
            erow_blk, erow_blk,
            pl.BlockSpec((BE, 8), lambda i: (i, 0)),
            wspec,
            pl.BlockSpec((8, D), full),
            pl.BlockSpec((1, D), full),
            wspec,
            pl.BlockSpec((1, D), full),
        ],
        out_specs=erow_blk,
        out_shape=jax.ShapeDtypeStruct((Epad, D), F32),
        compiler_params=cparams,
    )(g1e, mge, vec_p, g2_w1[D:2 * D], w2v_p, b1e, g2_w2, b2e)

    sc_scatter = _make_sc_scatter(Epad, EPW, NA, D)
    zsum = jnp.zeros((NA, D), F32)
    zcnt = jnp.zeros((NA, 16), F32)
    ones_rows = jnp.ones((CH, 16), F32)
    psums, pcnts = sc_scatter(contrib, dst_p, zsum, zcnt, ones_rows)

    out_p = pl.pallas_call(
        _tc3_body,
        grid=(Np // BN,),
        in_specs=[
            row_blk, row_blk2, row_blk,
            pl.BlockSpec((BN, 8), lambda i: (i, 0)),
            pl.BlockSpec((NC, BN, D), lambda i: (0, i, 0)),
            pl.BlockSpec((NC, BN, 16), lambda i: (0, i, 0)),
            pl.BlockSpec((8, D), full),
            pl.BlockSpec((1, D), full),
            wspec,
            pl.BlockSpec((1, D), full),
            wspec,
            pl.BlockSpec((1, D), full),
            wspec,
            pl.BlockSpec((1, D), full),
        ],
        out_specs=row_blk,
        out_shape=jax.ShapeDtypeStruct((Np, D), F32),
        compiler_params=cparams,
    )(res_p, gsec, q_res, pe_p, psums, pcnts,
      w1v_n, b1g, g1_w2, b2g, f_w1, fb1, f_w2, fb2)

    return out_p[:N]
